# baseline (device time: 31059 ns/iter reference)
import jax
import jax.numpy as jnp
from jax import lax
from jax.experimental import pallas as pl
from jax.experimental.pallas import tpu as pltpu

N_DEV = 4
N_LAYERS = 3
SEND_ORDER = (2, 1, 3)


def kernel(x, Win0, Wout0, Win1, Wout1, Win2, Wout2):
    b, d_in = x.shape
    h_dim = Win0.shape[1]
    d_out = Wout0.shape[1]

    def body(x_ref, win0_ref, wout0_ref, win1_ref, wout1_ref, win2_ref,
             wout2_ref, out_ref, win_v, wout_v, send_buf, comm_ref,
             w_sems, send_sems, recv_sems):
        my = lax.axis_index("i")

        wins_hbm = [win0_ref, win1_ref, win2_ref]
        wouts_hbm = [wout0_ref, wout1_ref, wout2_ref]
        w_copies = []
        for layer in range(N_LAYERS):
            cin = pltpu.make_async_copy(
                wins_hbm[layer], win_v.at[layer], w_sems.at[layer, 0])
            cout = pltpu.make_async_copy(
                wouts_hbm[layer], wout_v.at[layer], w_sems.at[layer, 1])
            cin.start()
            cout.start()
            w_copies.append((cin, cout))

        barrier_sem = pltpu.get_barrier_semaphore()
        for off in range(1, N_DEV):
            pl.semaphore_signal(
                barrier_sem, inc=1,
                device_id=((my + off) % N_DEV,),
                device_id_type=pl.DeviceIdType.MESH,
            )
        pl.semaphore_wait(barrier_sem, N_DEV - 1)

        xv = x_ref[...].astype(jnp.bfloat16)
        for layer in range(N_LAYERS):
            w_copies[layer][0].wait()
            partial = jnp.dot(
                xv, win_v[layer].astype(jnp.bfloat16),
                preferred_element_type=jnp.float32,
            )
            send_buf[layer] = partial.astype(jnp.bfloat16)

            rdmas = {}
            for off in SEND_ORDER:
                rdma = pltpu.make_async_remote_copy(
                    src_ref=send_buf.at[layer],
                    dst_ref=comm_ref.at[layer, off - 1],
                    send_sem=send_sems.at[layer, off - 1],
                    recv_sem=recv_sems.at[layer, off - 1],
                    device_id=((my + off) % N_DEV,),
                    device_id_type=pl.DeviceIdType.MESH,
                )
                rdma.start()
                rdmas[off] = rdma

            acc = partial
            for off in (1, 3, 2):
                rdmas[off].wait_recv()
                acc = acc + comm_ref[layer, off - 1].astype(jnp.float32)
            for off in SEND_ORDER:
                rdmas[off].wait_send()

            h = jnp.maximum(acc, 0.0).astype(jnp.bfloat16)
            w_copies[layer][1].wait()
            nxt = jnp.dot(
                h, wout_v[layer].astype(jnp.bfloat16),
                preferred_element_type=jnp.float32,
            )
            if layer == N_LAYERS - 1:
                out_ref[...] = nxt
            else:
                xv = nxt.astype(jnp.bfloat16)

    return pl.pallas_call(
        body,
        out_shape=jax.ShapeDtypeStruct((b, d_out), jnp.float32),
        in_specs=[pl.BlockSpec(memory_space=pltpu.VMEM)]
        + [pl.BlockSpec(memory_space=pl.ANY)] * 6,
        out_specs=pl.BlockSpec(memory_space=pltpu.VMEM),
        scratch_shapes=[
            pltpu.VMEM((N_LAYERS, d_in, h_dim), jnp.float32),
            pltpu.VMEM((N_LAYERS, h_dim, d_out), jnp.float32),
            pltpu.VMEM((N_LAYERS, b, h_dim), jnp.bfloat16),
            pltpu.VMEM((N_LAYERS, N_DEV - 1, b, h_dim), jnp.bfloat16),
            pltpu.SemaphoreType.DMA((N_LAYERS, 2)),
            pltpu.SemaphoreType.DMA((N_LAYERS, N_DEV - 1)),
            pltpu.SemaphoreType.DMA((N_LAYERS, N_DEV - 1)),
        ],
        compiler_params=pltpu.CompilerParams(collective_id=0),
    )(x, Win0, Wout0, Win1, Wout1, Win2, Wout2)


# device time: 31038 ns/iter; 1.0007x vs baseline; 1.0007x over previous
import jax
import jax.numpy as jnp
from jax import lax
from jax.experimental import pallas as pl
from jax.experimental.pallas import tpu as pltpu

N_DEV = 4
N_LAYERS = 3
SEND_ORDER = (2, 1, 3)


def kernel(x, Win0, Wout0, Win1, Wout1, Win2, Wout2):
    b, d_in = x.shape
    h_dim = Win0.shape[1]
    d_out = Wout0.shape[1]

    def body(x_ref, win0_ref, wout0_ref, win1_ref, wout1_ref, win2_ref,
             wout2_ref, out_ref, win_v, wout_v, send_buf, comm_ref,
             w_sems, send_sems, recv_sems):
        my = lax.axis_index("i")

        wins_hbm = [win0_ref, win1_ref, win2_ref]
        wouts_hbm = [wout0_ref, wout1_ref, wout2_ref]
        w_copies = []
        for layer in range(N_LAYERS):
            cin = pltpu.make_async_copy(
                wins_hbm[layer], win_v.at[layer], w_sems.at[layer, 0])
            cout = pltpu.make_async_copy(
                wouts_hbm[layer], wout_v.at[layer], w_sems.at[layer, 1])
            cin.start()
            cout.start()
            w_copies.append((cin, cout))

        barrier_sem = pltpu.get_barrier_semaphore()
        for off in range(1, N_DEV):
            pl.semaphore_signal(
                barrier_sem, inc=1,
                device_id=((my + off) % N_DEV,),
                device_id_type=pl.DeviceIdType.MESH,
            )
        pl.semaphore_wait(barrier_sem, N_DEV - 1)

        xv = x_ref[...].astype(jnp.bfloat16)
        for layer in range(N_LAYERS):
            w_copies[layer][0].wait()
            partial = jnp.dot(
                xv, win_v[layer].astype(jnp.bfloat16),
                preferred_element_type=jnp.float32,
            )
            send_buf[layer] = partial.astype(jnp.bfloat16)

            rdmas = {}
            for off in SEND_ORDER:
                rdma = pltpu.make_async_remote_copy(
                    src_ref=send_buf.at[layer],
                    dst_ref=comm_ref.at[layer, off - 1],
                    send_sem=send_sems.at[layer, off - 1],
                    recv_sem=recv_sems.at[layer, off - 1],
                    device_id=((my + off) % N_DEV,),
                    device_id_type=pl.DeviceIdType.MESH,
                )
                rdma.start()
                rdmas[off] = rdma

            acc = partial
            for off in (1, 3, 2):
                rdmas[off].wait_recv()
                acc = acc + comm_ref[layer, off - 1].astype(jnp.float32)
            for off in SEND_ORDER:
                rdmas[off].wait_send()

            h = jnp.maximum(acc, 0.0).astype(jnp.bfloat16)
            w_copies[layer][1].wait()
            nxt = jnp.dot(
                h, wout_v[layer].astype(jnp.bfloat16),
                preferred_element_type=jnp.float32,
            )
            if layer == N_LAYERS - 1:
                out_ref[...] = nxt
            else:
                xv = nxt.astype(jnp.bfloat16)

    return pl.pallas_call(
        body,
        out_shape=jax.ShapeDtypeStruct((b, d_out), jnp.float32),
        in_specs=[pl.BlockSpec(memory_space=pltpu.VMEM)]
        + [pl.BlockSpec(memory_space=pltpu.MemorySpace.HBM)] * 6,
        out_specs=pl.BlockSpec(memory_space=pltpu.VMEM),
        scratch_shapes=[
            pltpu.VMEM((N_LAYERS, d_in, h_dim), jnp.float32),
            pltpu.VMEM((N_LAYERS, h_dim, d_out), jnp.float32),
            pltpu.VMEM((N_LAYERS, b, h_dim), jnp.bfloat16),
            pltpu.VMEM((N_LAYERS, N_DEV - 1, b, h_dim), jnp.bfloat16),
            pltpu.SemaphoreType.DMA((N_LAYERS, 2)),
            pltpu.SemaphoreType.DMA((N_LAYERS, N_DEV - 1)),
            pltpu.SemaphoreType.DMA((N_LAYERS, N_DEV - 1)),
        ],
        compiler_params=pltpu.CompilerParams(collective_id=0),
    )(x, Win0, Wout0, Win1, Wout1, Win2, Wout2)


# device time: 28893 ns/iter; 1.0750x vs baseline; 1.0742x over previous
import jax
import jax.numpy as jnp
from jax import lax
from jax.experimental import pallas as pl
from jax.experimental.pallas import tpu as pltpu

N_DEV = 4
N_LAYERS = 3
SEND_ORDER = (2, 1, 3)


def kernel(x, Win0, Wout0, Win1, Wout1, Win2, Wout2):
    b, d_in = x.shape
    h_dim = Win0.shape[1]
    d_out = Wout0.shape[1]

    def body(x_ref, win0_ref, wout0_ref, win1_ref, wout1_ref, win2_ref,
             wout2_ref, out_ref, send_buf, comm_ref, send_sems, recv_sems):
        my = lax.axis_index("i")

        barrier_sem = pltpu.get_barrier_semaphore()
        for off in range(1, N_DEV):
            pl.semaphore_signal(
                barrier_sem, inc=1,
                device_id=((my + off) % N_DEV,),
                device_id_type=pl.DeviceIdType.MESH,
            )
        pl.semaphore_wait(barrier_sem, N_DEV - 1)

        wins = [win0_ref, win1_ref, win2_ref]
        wouts = [wout0_ref, wout1_ref, wout2_ref]

        xv = x_ref[...]
        for layer in range(N_LAYERS):
            partial = jnp.dot(
                xv, wins[layer][...], preferred_element_type=jnp.float32,
            )
            send_buf[layer] = partial.astype(jnp.bfloat16)

            rdmas = {}
            for off in SEND_ORDER:
                rdma = pltpu.make_async_remote_copy(
                    src_ref=send_buf.at[layer],
                    dst_ref=comm_ref.at[layer, off - 1],
                    send_sem=send_sems.at[layer, off - 1],
                    recv_sem=recv_sems.at[layer, off - 1],
                    device_id=((my + off) % N_DEV,),
                    device_id_type=pl.DeviceIdType.MESH,
                )
                rdma.start()
                rdmas[off] = rdma

            acc = partial
            for off in (1, 3, 2):
                rdmas[off].wait_recv()
                acc = acc + comm_ref[layer, off - 1].astype(jnp.float32)
            for off in SEND_ORDER:
                rdmas[off].wait_send()

            h = jnp.maximum(acc, 0.0).astype(jnp.bfloat16)
            nxt = jnp.dot(
                h, wouts[layer][...], preferred_element_type=jnp.float32,
            )
            if layer == N_LAYERS - 1:
                out_ref[...] = nxt
            else:
                xv = nxt.astype(jnp.bfloat16)

    args = [a.astype(jnp.bfloat16)
            for a in (x, Win0, Wout0, Win1, Wout1, Win2, Wout2)]
    return pl.pallas_call(
        body,
        out_shape=jax.ShapeDtypeStruct((b, d_out), jnp.float32),
        in_specs=[pl.BlockSpec(memory_space=pltpu.VMEM)] * 7,
        out_specs=pl.BlockSpec(memory_space=pltpu.VMEM),
        scratch_shapes=[
            pltpu.VMEM((N_LAYERS, b, h_dim), jnp.bfloat16),
            pltpu.VMEM((N_LAYERS, N_DEV - 1, b, h_dim), jnp.bfloat16),
            pltpu.SemaphoreType.DMA((N_LAYERS, N_DEV - 1)),
            pltpu.SemaphoreType.DMA((N_LAYERS, N_DEV - 1)),
        ],
        compiler_params=pltpu.CompilerParams(collective_id=0),
    )(*args)


# device time: 28386 ns/iter; 1.0942x vs baseline; 1.0179x over previous
import jax
import jax.numpy as jnp
from jax import lax
from jax.experimental import pallas as pl
from jax.experimental.pallas import tpu as pltpu

N_DEV = 4
N_LAYERS = 3
SEND_ORDER = (2, 1, 3)


def kernel(x, Win0, Wout0, Win1, Wout1, Win2, Wout2):
    b, d_in = x.shape
    h_dim = Win0.shape[1]
    d_out = Wout0.shape[1]

    def body(x_ref, win0_ref, wout0_ref, win1_ref, wout1_ref, win2_ref,
             wout2_ref, out_ref, send_buf, comm_ref, send_sems, recv_sems):
        my = lax.axis_index("i")

        barrier_sem = pltpu.get_barrier_semaphore()
        for off in range(1, N_DEV):
            pl.semaphore_signal(
                barrier_sem, inc=1,
                device_id=((my + off) % N_DEV,),
                device_id_type=pl.DeviceIdType.MESH,
            )

        wins = [win0_ref, win1_ref, win2_ref]
        wouts = [wout0_ref, wout1_ref, wout2_ref]

        xv = x_ref[...]
        for layer in range(N_LAYERS):
            partial = jnp.dot(
                xv, wins[layer][...], preferred_element_type=jnp.float32,
            )
            pb = partial.astype(jnp.bfloat16)
            send_buf[layer] = pb
            if layer == 0:
                pl.semaphore_wait(barrier_sem, N_DEV - 1)

            rdmas = {}
            for off in SEND_ORDER:
                rdma = pltpu.make_async_remote_copy(
                    src_ref=send_buf.at[layer],
                    dst_ref=comm_ref.at[layer, off - 1],
                    send_sem=send_sems.at[layer, off - 1],
                    recv_sem=recv_sems.at[layer, off - 1],
                    device_id=((my + off) % N_DEV,),
                    device_id_type=pl.DeviceIdType.MESH,
                )
                rdma.start()
                rdmas[off] = rdma

            acc = pb
            for off in (1, 3, 2):
                rdmas[off].wait_recv()
                acc = acc + comm_ref[layer, off - 1]
            for off in SEND_ORDER:
                rdmas[off].wait_send()

            h = jnp.maximum(acc, jnp.bfloat16(0.0))
            nxt = jnp.dot(
                h, wouts[layer][...], preferred_element_type=jnp.float32,
            )
            if layer == N_LAYERS - 1:
                out_ref[...] = nxt
            else:
                xv = nxt.astype(jnp.bfloat16)

    args = [a.astype(jnp.bfloat16)
            for a in (x, Win0, Wout0, Win1, Wout1, Win2, Wout2)]
    return pl.pallas_call(
        body,
        out_shape=jax.ShapeDtypeStruct((b, d_out), jnp.float32),
        in_specs=[pl.BlockSpec(memory_space=pltpu.VMEM)] * 7,
        out_specs=pl.BlockSpec(memory_space=pltpu.VMEM),
        scratch_shapes=[
            pltpu.VMEM((N_LAYERS, b, h_dim), jnp.bfloat16),
            pltpu.VMEM((N_LAYERS, N_DEV - 1, b, h_dim), jnp.bfloat16),
            pltpu.SemaphoreType.DMA((N_LAYERS, N_DEV - 1)),
            pltpu.SemaphoreType.DMA((N_LAYERS, N_DEV - 1)),
        ],
        compiler_params=pltpu.CompilerParams(collective_id=0),
    )(*args)


# device time: 26455 ns/iter; 1.1740x vs baseline; 1.0730x over previous
import jax
import jax.numpy as jnp
from jax import lax
from jax.experimental import pallas as pl
from jax.experimental.pallas import tpu as pltpu

N_DEV = 4
N_LAYERS = 3
SEND_ORDER = (2, 1, 3)


def kernel(x, Win0, Wout0, Win1, Wout1, Win2, Wout2):
    b, d_in = x.shape
    h_dim = Win0.shape[1]
    d_out = Wout0.shape[1]

    def body(x_ref, wins_ref, wouts_ref, out_ref, send_buf, comm_ref,
             send_sems, recv_sems):
        my = lax.axis_index("i")

        barrier_sem = pltpu.get_barrier_semaphore()
        for off in range(1, N_DEV):
            pl.semaphore_signal(
                barrier_sem, inc=1,
                device_id=((my + off) % N_DEV,),
                device_id_type=pl.DeviceIdType.MESH,
            )

        xv = x_ref[...].astype(jnp.bfloat16)
        for layer in range(N_LAYERS):
            partial = jnp.dot(
                xv, wins_ref[layer], preferred_element_type=jnp.float32,
            )
            pb = partial.astype(jnp.bfloat16)
            send_buf[layer] = pb
            if layer == 0:
                pl.semaphore_wait(barrier_sem, N_DEV - 1)

            rdmas = {}
            for off in SEND_ORDER:
                rdma = pltpu.make_async_remote_copy(
                    src_ref=send_buf.at[layer],
                    dst_ref=comm_ref.at[layer, off - 1],
                    send_sem=send_sems.at[layer, off - 1],
                    recv_sem=recv_sems.at[layer, off - 1],
                    device_id=((my + off) % N_DEV,),
                    device_id_type=pl.DeviceIdType.MESH,
                )
                rdma.start()
                rdmas[off] = rdma

            acc = pb
            for off in (1, 3, 2):
                rdmas[off].wait_recv()
                acc = acc + comm_ref[layer, off - 1]
            for off in SEND_ORDER:
                rdmas[off].wait_send()

            h = jnp.maximum(acc, jnp.bfloat16(0.0))
            nxt = jnp.dot(
                h, wouts_ref[layer], preferred_element_type=jnp.float32,
            )
            if layer == N_LAYERS - 1:
                out_ref[...] = nxt
            else:
                xv = nxt.astype(jnp.bfloat16)

    wins = jnp.stack([Win0, Win1, Win2]).astype(jnp.bfloat16)
    wouts = jnp.stack([Wout0, Wout1, Wout2]).astype(jnp.bfloat16)
    return pl.pallas_call(
        body,
        out_shape=jax.ShapeDtypeStruct((b, d_out), jnp.float32),
        in_specs=[pl.BlockSpec(memory_space=pltpu.VMEM)] * 3,
        out_specs=pl.BlockSpec(memory_space=pltpu.VMEM),
        scratch_shapes=[
            pltpu.VMEM((N_LAYERS, b, h_dim), jnp.bfloat16),
            pltpu.VMEM((N_LAYERS, N_DEV - 1, b, h_dim), jnp.bfloat16),
            pltpu.SemaphoreType.DMA((N_LAYERS, N_DEV - 1)),
            pltpu.SemaphoreType.DMA((N_LAYERS, N_DEV - 1)),
        ],
        compiler_params=pltpu.CompilerParams(collective_id=0),
    )(x, wins, wouts)
